# Initial kernel scaffold; baseline (speedup 1.0000x reference)
import jax
import jax.numpy as jnp
from jax.experimental import pallas as pl


def kernel(z, embeddings):
    raise NotImplementedError("write your pallas kernel here")


# speedup vs baseline: 1.4820x; 1.4820x over previous
import functools

import jax
import jax.numpy as jnp
from jax import lax
from jax.experimental import pallas as pl
from jax.experimental.pallas import tpu as pltpu
from jax.experimental.pallas import tpu_sc as plsc

G = 8
KD = 64
C = 8192
N = 16
T = 1024
CT = 1024
NCT = C // CT
B = G * N * T


KA = 72


def _prep_body(emb_ref, embt_ref, aug_ref):
    e = emb_ref[0]
    et = e.T
    embt_ref[0] = et
    etm2 = (et * -2.0).astype(jnp.bfloat16)
    e2 = jnp.sum(et * et, axis=1, keepdims=True)
    h1 = e2.astype(jnp.bfloat16)
    r1 = e2 - h1.astype(jnp.float32)
    h2 = r1.astype(jnp.bfloat16)
    r2 = r1 - h2.astype(jnp.float32)
    h3 = r2.astype(jnp.bfloat16)
    zpad = jnp.zeros((CT, KA - KD - 3), jnp.bfloat16)
    aug_ref[0] = jnp.concatenate([etm2, h1, h2, h3, zpad], axis=1)


def _prep(embeddings):
    return pl.pallas_call(
        _prep_body,
        grid=(G, NCT),
        in_specs=[pl.BlockSpec((1, KD, CT), lambda g, c: (g, 0, c))],
        out_specs=[
            pl.BlockSpec((1, CT, KD), lambda g, c: (g, c, 0)),
            pl.BlockSpec((1, CT, KA), lambda g, c: (g, c, 0)),
        ],
        out_shape=[
            jax.ShapeDtypeStruct((G, C, KD), jnp.float32),
            jax.ShapeDtypeStruct((G, C, KA), jnp.bfloat16),
        ],
    )(embeddings)


def _argmin_body(x_ref, aug_ref, idx_ref, minv, amin):
    g = pl.program_id(0)
    ct = pl.program_id(2)
    xb = x_ref[0, 0]
    aug = aug_ref[0]
    onepad = jnp.where(
        lax.broadcasted_iota(jnp.int32, (KA - KD, T), 0) < 3,
        1.0, 0.0).astype(jnp.bfloat16)
    xa = jnp.concatenate([xb, onepad], axis=0)
    scores = lax.dot_general(aug, xa, (((1,), (0,)), ((), ())),
                             preferred_element_type=jnp.float32)
    m = jnp.min(scores, axis=0)
    cidx = lax.broadcasted_iota(jnp.int32, (CT, T), 0).astype(jnp.float32)
    cand = jnp.where(scores == m[None, :], cidx, jnp.float32(C))
    targ = jnp.min(cand, axis=0)

    @pl.when(ct == 0)
    def _():
        minv[0, :] = m
        amin[0, :] = targ

    @pl.when(ct != 0)
    def _():
        prev = minv[0, :]
        upd = m < prev
        minv[0, :] = jnp.where(upd, m, prev)
        amin[0, :] = jnp.where(upd, targ + ct * CT, amin[0, :])

    @pl.when(ct == NCT - 1)
    def _():
        idx_ref[0, 0, 0, :] = (amin[0, :] + g * C).astype(jnp.int32)


def _argmin(zb, aug):
    return pl.pallas_call(
        _argmin_body,
        grid=(G, N, NCT),
        in_specs=[
            pl.BlockSpec((1, 1, KD, T), lambda g, n, c: (n, g, 0, 0)),
            pl.BlockSpec((1, CT, KA), lambda g, n, c: (g, c, 0)),
        ],
        out_specs=pl.BlockSpec((1, 1, 1, T), lambda g, n, c: (g, n, 0, 0)),
        out_shape=jax.ShapeDtypeStruct((G, N, 1, T), jnp.int32),
        scratch_shapes=[
            pltpu.VMEM((1, T), jnp.float32),
            pltpu.VMEM((1, T), jnp.float32),
        ],
    )(zb, aug)


_NW = 32
_BPW = B // _NW
_CH = 128


def _gather(table, idx_flat):
    mesh = plsc.VectorSubcoreMesh(core_axis_name="c", subcore_axis_name="s")

    @functools.partial(
        pl.kernel,
        mesh=mesh,
        compiler_params=pltpu.CompilerParams(use_tc_tiling_on_sc=False),
        out_type=jax.ShapeDtypeStruct((B, KD), jnp.float32),
        scratch_types=[
            pltpu.VMEM((_CH,), jnp.int32),
            pltpu.VMEM((_CH, KD), jnp.float32),
            pltpu.SemaphoreType.DMA,
        ],
    )
    def gk(tab, idx, out, idx_v, rows_v, sem):
        wid = lax.axis_index("s") * 2 + lax.axis_index("c")
        base = wid * _BPW

        def body(i, carry):
            off = base + i * _CH
            pltpu.sync_copy(idx.at[pl.ds(off, _CH)], idx_v)
            pltpu.async_copy(tab.at[idx_v], rows_v, sem).wait()
            pltpu.sync_copy(rows_v, out.at[pl.ds(off, _CH)])
            return carry

        lax.fori_loop(0, _BPW // _CH, body, 0)

    return gk(table, idx_flat)


def _asm_body(q_ref, x_ref, out_ref, loss_ref):
    nidx = pl.program_id(0)
    gidx = pl.program_id(1)
    q = q_ref[0, 0]
    x = x_ref[0, 0]
    qt = q.T
    d = qt - x
    out_ref[0, 0] = x + d

    @pl.when(gidx == 0)
    def _():
        loss_ref[nidx, 0] = 0.0

    loss_ref[nidx, 0] += jnp.sum(d * d)


def _assemble(q4, zr):
    return pl.pallas_call(
        _asm_body,
        grid=(N, G),
        in_specs=[
            pl.BlockSpec((1, 1, T, KD), lambda n, g: (g, n, 0, 0)),
            pl.BlockSpec((1, 1, KD, T), lambda n, g: (n, g, 0, 0)),
        ],
        out_specs=[
            pl.BlockSpec((1, 1, KD, T), lambda n, g: (n, g, 0, 0)),
            pl.BlockSpec((N, 1), lambda n, g: (0, 0),
                         memory_space=pltpu.SMEM),
        ],
        out_shape=[
            jax.ShapeDtypeStruct((N, G, KD, T), jnp.float32),
            jax.ShapeDtypeStruct((N, 1), jnp.float32),
        ],
    )(q4, zr)


def kernel(z, embeddings):
    zr = z.reshape(N, G, KD, T)
    zb = zr.astype(jnp.bfloat16)
    embt, aug = _prep(embeddings)
    idx = _argmin(zb, aug)
    q = _gather(embt.reshape(G * C, KD), idx.reshape(B))
    out4, loss = _assemble(q.reshape(G, N, T, KD), zr)
    q_merge = out4.reshape(N, 128, 4096)
    vq_loss = loss[:, 0] * (0.25 / (KD * T * G))
    return (q_merge, vq_loss)


# speedup vs baseline: 1.7755x; 1.1981x over previous
import functools

import jax
import jax.numpy as jnp
from jax import lax
from jax.experimental import pallas as pl
from jax.experimental.pallas import tpu as pltpu
from jax.experimental.pallas import tpu_sc as plsc

G = 8
KD = 64
C = 8192
N = 16
T = 1024
CT = 1024
NCT = C // CT
B = G * N * T


KA = 72


def _prep_body(emb_ref, embt_ref, aug_ref):
    e = emb_ref[0]
    et = e.T
    embt_ref[0] = et
    etm2 = (et * -2.0).astype(jnp.bfloat16)
    e2 = jnp.sum(et * et, axis=1, keepdims=True)
    h1 = e2.astype(jnp.bfloat16)
    r1 = e2 - h1.astype(jnp.float32)
    h2 = r1.astype(jnp.bfloat16)
    r2 = r1 - h2.astype(jnp.float32)
    h3 = r2.astype(jnp.bfloat16)
    zpad = jnp.zeros((CT, KA - KD - 3), jnp.bfloat16)
    aug_ref[0] = jnp.concatenate([etm2, h1, h2, h3, zpad], axis=1)


def _prep(embeddings):
    return pl.pallas_call(
        _prep_body,
        grid=(G, NCT),
        in_specs=[pl.BlockSpec((1, KD, CT), lambda g, c: (g, 0, c))],
        out_specs=[
            pl.BlockSpec((1, CT, KD), lambda g, c: (g, c, 0)),
            pl.BlockSpec((1, CT, KA), lambda g, c: (g, c, 0)),
        ],
        out_shape=[
            jax.ShapeDtypeStruct((G, C, KD), jnp.float32),
            jax.ShapeDtypeStruct((G, C, KA), jnp.bfloat16),
        ],
    )(embeddings)


def _argmin_body(x_ref, aug_ref, idx_ref):
    g = pl.program_id(0)
    xb = x_ref[0, 0]
    aug = aug_ref[0]
    onepad = jnp.where(
        lax.broadcasted_iota(jnp.int32, (KA - KD, T), 0) < 3,
        1.0, 0.0).astype(jnp.bfloat16)
    xa = jnp.concatenate([xb, onepad], axis=0)
    scores = lax.dot_general(aug, xa, (((1,), (0,)), ((), ())),
                             preferred_element_type=jnp.float32)
    m = jnp.min(scores, axis=0)
    cidx = lax.broadcasted_iota(jnp.int32, (C, T), 0).astype(jnp.float32)
    cand = jnp.where(scores == m[None, :], cidx, jnp.float32(C))
    targ = jnp.min(cand, axis=0)
    idx_ref[0, 0, 0, :] = (targ + g * C).astype(jnp.int32)


def _argmin(zb, aug):
    return pl.pallas_call(
        _argmin_body,
        grid=(G, N),
        in_specs=[
            pl.BlockSpec((1, 1, KD, T), lambda g, n: (n, g, 0, 0)),
            pl.BlockSpec((1, C, KA), lambda g, n: (g, 0, 0)),
        ],
        out_specs=pl.BlockSpec((1, 1, 1, T), lambda g, n: (g, n, 0, 0)),
        out_shape=jax.ShapeDtypeStruct((G, N, 1, T), jnp.int32),
    )(zb, aug)


_NW = 32
_BPW = B // _NW
_CH = 128


def _gather(table, idx_flat):
    mesh = plsc.VectorSubcoreMesh(core_axis_name="c", subcore_axis_name="s")

    @functools.partial(
        pl.kernel,
        mesh=mesh,
        compiler_params=pltpu.CompilerParams(use_tc_tiling_on_sc=False),
        out_type=jax.ShapeDtypeStruct((B, KD), jnp.float32),
        scratch_types=[
            pltpu.VMEM((_CH,), jnp.int32),
            pltpu.VMEM((_CH, KD), jnp.float32),
            pltpu.SemaphoreType.DMA,
        ],
    )
    def gk(tab, idx, out, idx_v, rows_v, sem):
        wid = lax.axis_index("s") * 2 + lax.axis_index("c")
        base = wid * _BPW

        def body(i, carry):
            off = base + i * _CH
            pltpu.sync_copy(idx.at[pl.ds(off, _CH)], idx_v)
            pltpu.async_copy(tab.at[idx_v], rows_v, sem).wait()
            pltpu.sync_copy(rows_v, out.at[pl.ds(off, _CH)])
            return carry

        lax.fori_loop(0, _BPW // _CH, body, 0)

    return gk(table, idx_flat)


def _asm_body(q_ref, x_ref, out_ref, loss_ref):
    nidx = pl.program_id(0)
    gidx = pl.program_id(1)
    q = q_ref[0, 0]
    x = x_ref[0, 0]
    qt = q.T
    d = qt - x
    out_ref[0, 0] = x + d

    @pl.when(gidx == 0)
    def _():
        loss_ref[nidx, 0] = 0.0

    loss_ref[nidx, 0] += jnp.sum(d * d)


def _assemble(q4, zr):
    return pl.pallas_call(
        _asm_body,
        grid=(N, G),
        in_specs=[
            pl.BlockSpec((1, 1, T, KD), lambda n, g: (g, n, 0, 0)),
            pl.BlockSpec((1, 1, KD, T), lambda n, g: (n, g, 0, 0)),
        ],
        out_specs=[
            pl.BlockSpec((1, 1, KD, T), lambda n, g: (n, g, 0, 0)),
            pl.BlockSpec((N, 1), lambda n, g: (0, 0),
                         memory_space=pltpu.SMEM),
        ],
        out_shape=[
            jax.ShapeDtypeStruct((N, G, KD, T), jnp.float32),
            jax.ShapeDtypeStruct((N, 1), jnp.float32),
        ],
    )(q4, zr)


def kernel(z, embeddings):
    zr = z.reshape(N, G, KD, T)
    zb = zr.astype(jnp.bfloat16)
    embt, aug = _prep(embeddings)
    idx = _argmin(zb, aug)
    q = _gather(embt.reshape(G * C, KD), idx.reshape(B))
    out4, loss = _assemble(q.reshape(G, N, T, KD), zr)
    q_merge = out4.reshape(N, 128, 4096)
    vq_loss = loss[:, 0] * (0.25 / (KD * T * G))
    return (q_merge, vq_loss)


# speedup vs baseline: 1.9287x; 1.0863x over previous
import functools

import jax
import jax.numpy as jnp
from jax import lax
from jax.experimental import pallas as pl
from jax.experimental.pallas import tpu as pltpu
from jax.experimental.pallas import tpu_sc as plsc

G = 8
KD = 64
C = 8192
N = 16
T = 1024
CT = 1024
NCT = C // CT
B = G * N * T


KA = 72


def _prep_body(emb_ref, embt_ref, aug_ref):
    e = emb_ref[0]
    et = e.T
    embt_ref[0] = et
    etm2 = (et * -2.0).astype(jnp.bfloat16)
    e2 = jnp.sum(et * et, axis=1, keepdims=True)
    h1 = e2.astype(jnp.bfloat16)
    r1 = e2 - h1.astype(jnp.float32)
    h2 = r1.astype(jnp.bfloat16)
    r2 = r1 - h2.astype(jnp.float32)
    h3 = r2.astype(jnp.bfloat16)
    zpad = jnp.zeros((CT, KA - KD - 3), jnp.bfloat16)
    aug_ref[0] = jnp.concatenate([etm2, h1, h2, h3, zpad], axis=1)


def _prep(embeddings):
    return pl.pallas_call(
        _prep_body,
        grid=(G, NCT),
        in_specs=[pl.BlockSpec((1, KD, CT), lambda g, c: (g, 0, c))],
        out_specs=[
            pl.BlockSpec((1, CT, KD), lambda g, c: (g, c, 0)),
            pl.BlockSpec((1, CT, KA), lambda g, c: (g, c, 0)),
        ],
        out_shape=[
            jax.ShapeDtypeStruct((G, C, KD), jnp.float32),
            jax.ShapeDtypeStruct((G, C, KA), jnp.bfloat16),
        ],
    )(embeddings)


NCH = 8
CH = C // NCH


def _argmin_body(x_ref, aug_ref, idx_ref):
    g = pl.program_id(0)
    xb = x_ref[0, 0]
    onepad = jnp.where(
        lax.broadcasted_iota(jnp.int32, (KA - KD, T), 0) < 3,
        1.0, 0.0).astype(jnp.bfloat16)
    xa = jnp.concatenate([xb, onepad], axis=0)
    run_m = None
    run_i = None
    for c in range(NCH):
        a = aug_ref[0, c * CH:(c + 1) * CH, :]
        s = lax.dot_general(a, xa, (((1,), (0,)), ((), ())),
                            preferred_element_type=jnp.float32)
        mc = jnp.min(s, axis=0)
        cidx = (lax.broadcasted_iota(jnp.int32, (CH, T), 0)
                + c * CH).astype(jnp.float32)
        tc = jnp.min(jnp.where(s == mc[None, :], cidx, jnp.float32(C)),
                     axis=0)
        if c == 0:
            run_m, run_i = mc, tc
        else:
            upd = mc < run_m
            run_m = jnp.where(upd, mc, run_m)
            run_i = jnp.where(upd, tc, run_i)
    idx_ref[0, 0, 0, :] = (run_i + g * C).astype(jnp.int32)


def _argmin(zb, aug):
    return pl.pallas_call(
        _argmin_body,
        grid=(G, N),
        in_specs=[
            pl.BlockSpec((1, 1, KD, T), lambda g, n: (n, g, 0, 0)),
            pl.BlockSpec((1, C, KA), lambda g, n: (g, 0, 0)),
        ],
        out_specs=pl.BlockSpec((1, 1, 1, T), lambda g, n: (g, n, 0, 0)),
        out_shape=jax.ShapeDtypeStruct((G, N, 1, T), jnp.int32),
    )(zb, aug)


_NW = 32
_BPW = B // _NW
_CH = 128


def _gather(table, idx_flat):
    mesh = plsc.VectorSubcoreMesh(core_axis_name="c", subcore_axis_name="s")

    @functools.partial(
        pl.kernel,
        mesh=mesh,
        compiler_params=pltpu.CompilerParams(use_tc_tiling_on_sc=False),
        out_type=jax.ShapeDtypeStruct((B, KD), jnp.float32),
        scratch_types=[
            pltpu.VMEM((_CH,), jnp.int32),
            pltpu.VMEM((_CH, KD), jnp.float32),
            pltpu.SemaphoreType.DMA,
        ],
    )
    def gk(tab, idx, out, idx_v, rows_v, sem):
        wid = lax.axis_index("s") * 2 + lax.axis_index("c")
        base = wid * _BPW

        def body(i, carry):
            off = base + i * _CH
            pltpu.sync_copy(idx.at[pl.ds(off, _CH)], idx_v)
            pltpu.async_copy(tab.at[idx_v], rows_v, sem).wait()
            pltpu.sync_copy(rows_v, out.at[pl.ds(off, _CH)])
            return carry

        lax.fori_loop(0, _BPW // _CH, body, 0)

    return gk(table, idx_flat)


def _asm_body(q_ref, x_ref, out_ref, loss_ref):
    nidx = pl.program_id(0)
    gidx = pl.program_id(1)
    q = q_ref[0, 0]
    x = x_ref[0, 0]
    qt = q.T
    d = qt - x
    out_ref[0, 0] = x + d

    @pl.when(gidx == 0)
    def _():
        loss_ref[nidx, 0] = 0.0

    loss_ref[nidx, 0] += jnp.sum(d * d)


def _assemble(q4, zr):
    return pl.pallas_call(
        _asm_body,
        grid=(N, G),
        in_specs=[
            pl.BlockSpec((1, 1, T, KD), lambda n, g: (g, n, 0, 0)),
            pl.BlockSpec((1, 1, KD, T), lambda n, g: (n, g, 0, 0)),
        ],
        out_specs=[
            pl.BlockSpec((1, 1, KD, T), lambda n, g: (n, g, 0, 0)),
            pl.BlockSpec((N, 1), lambda n, g: (0, 0),
                         memory_space=pltpu.SMEM),
        ],
        out_shape=[
            jax.ShapeDtypeStruct((N, G, KD, T), jnp.float32),
            jax.ShapeDtypeStruct((N, 1), jnp.float32),
        ],
    )(q4, zr)


def kernel(z, embeddings):
    zr = z.reshape(N, G, KD, T)
    zb = zr.astype(jnp.bfloat16)
    embt, aug = _prep(embeddings)
    idx = _argmin(zb, aug)
    q = _gather(embt.reshape(G * C, KD), idx.reshape(B))
    out4, loss = _assemble(q.reshape(G, N, T, KD), zr)
    q_merge = out4.reshape(N, 128, 4096)
    vq_loss = loss[:, 0] * (0.25 / (KD * T * G))
    return (q_merge, vq_loss)


# speedup vs baseline: 2.5205x; 1.3068x over previous
import functools

import jax
import jax.numpy as jnp
from jax import lax
from jax.experimental import pallas as pl
from jax.experimental.pallas import tpu as pltpu
from jax.experimental.pallas import tpu_sc as plsc

G = 8
KD = 64
C = 8192
N = 16
T = 1024
CT = 1024
NCT = C // CT
B = G * N * T


KA = 72


def _prep_body(emb_ref, embt_ref, aug_ref):
    e = emb_ref[0]
    et = e.T
    embt_ref[0] = et
    etm2 = (et * -2.0).astype(jnp.bfloat16)
    e2 = jnp.sum(et * et, axis=1, keepdims=True)
    h1 = e2.astype(jnp.bfloat16)
    r1 = e2 - h1.astype(jnp.float32)
    h2 = r1.astype(jnp.bfloat16)
    r2 = r1 - h2.astype(jnp.float32)
    h3 = r2.astype(jnp.bfloat16)
    zpad = jnp.zeros((CT, KA - KD - 3), jnp.bfloat16)
    aug_ref[0] = jnp.concatenate([etm2, h1, h2, h3, zpad], axis=1)


def _prep(embeddings):
    return pl.pallas_call(
        _prep_body,
        grid=(G, NCT),
        in_specs=[pl.BlockSpec((1, KD, CT), lambda g, c: (g, 0, c))],
        out_specs=[
            pl.BlockSpec((1, CT, KD), lambda g, c: (g, c, 0)),
            pl.BlockSpec((1, CT, KA), lambda g, c: (g, c, 0)),
        ],
        out_shape=[
            jax.ShapeDtypeStruct((G, C, KD), jnp.float32),
            jax.ShapeDtypeStruct((G, C, KA), jnp.bfloat16),
        ],
    )(embeddings)


NCH = 8
CH = C // NCH


def _argmin_body(x_ref, aug_ref, idx_ref):
    g = pl.program_id(0)
    xb = x_ref[0, 0]
    onepad = jnp.where(
        lax.broadcasted_iota(jnp.int32, (KA - KD, T), 0) < 3,
        1.0, 0.0).astype(jnp.bfloat16)
    xa = jnp.concatenate([xb, onepad], axis=0)
    run_m = None
    run_i = None
    for c in range(NCH):
        a = aug_ref[0, c * CH:(c + 1) * CH, :]
        s = lax.dot_general(a, xa, (((1,), (0,)), ((), ())),
                            preferred_element_type=jnp.float32)
        mc = jnp.min(s, axis=0)
        tc = (jnp.argmin(s, axis=0) + c * CH).astype(jnp.float32)
        if c == 0:
            run_m, run_i = mc, tc
        else:
            upd = mc < run_m
            run_m = jnp.where(upd, mc, run_m)
            run_i = jnp.where(upd, tc, run_i)
    idx_ref[0, 0, 0, :] = (run_i + g * C).astype(jnp.int32)


def _argmin(zb, aug):
    return pl.pallas_call(
        _argmin_body,
        grid=(G, N),
        in_specs=[
            pl.BlockSpec((1, 1, KD, T), lambda g, n: (n, g, 0, 0)),
            pl.BlockSpec((1, C, KA), lambda g, n: (g, 0, 0)),
        ],
        out_specs=pl.BlockSpec((1, 1, 1, T), lambda g, n: (g, n, 0, 0)),
        out_shape=jax.ShapeDtypeStruct((G, N, 1, T), jnp.int32),
    )(zb, aug)


_NW = 32
_BPW = B // _NW
_CH = 128


def _gather(table, idx_flat):
    mesh = plsc.VectorSubcoreMesh(core_axis_name="c", subcore_axis_name="s")

    @functools.partial(
        pl.kernel,
        mesh=mesh,
        compiler_params=pltpu.CompilerParams(use_tc_tiling_on_sc=False),
        out_type=jax.ShapeDtypeStruct((B, KD), jnp.float32),
        scratch_types=[
            pltpu.VMEM((_CH,), jnp.int32),
            pltpu.VMEM((_CH, KD), jnp.float32),
            pltpu.SemaphoreType.DMA,
        ],
    )
    def gk(tab, idx, out, idx_v, rows_v, sem):
        wid = lax.axis_index("s") * 2 + lax.axis_index("c")
        base = wid * _BPW

        def body(i, carry):
            off = base + i * _CH
            pltpu.sync_copy(idx.at[pl.ds(off, _CH)], idx_v)
            pltpu.async_copy(tab.at[idx_v], rows_v, sem).wait()
            pltpu.sync_copy(rows_v, out.at[pl.ds(off, _CH)])
            return carry

        lax.fori_loop(0, _BPW // _CH, body, 0)

    return gk(table, idx_flat)


def _asm_body(q_ref, x_ref, out_ref, loss_ref):
    nidx = pl.program_id(0)
    gidx = pl.program_id(1)
    q = q_ref[0, 0]
    x = x_ref[0, 0]
    qt = q.T
    d = qt - x
    out_ref[0, 0] = x + d

    @pl.when(gidx == 0)
    def _():
        loss_ref[nidx, 0] = 0.0

    loss_ref[nidx, 0] += jnp.sum(d * d)


def _assemble(q4, zr):
    return pl.pallas_call(
        _asm_body,
        grid=(N, G),
        in_specs=[
            pl.BlockSpec((1, 1, T, KD), lambda n, g: (g, n, 0, 0)),
            pl.BlockSpec((1, 1, KD, T), lambda n, g: (n, g, 0, 0)),
        ],
        out_specs=[
            pl.BlockSpec((1, 1, KD, T), lambda n, g: (n, g, 0, 0)),
            pl.BlockSpec((N, 1), lambda n, g: (0, 0),
                         memory_space=pltpu.SMEM),
        ],
        out_shape=[
            jax.ShapeDtypeStruct((N, G, KD, T), jnp.float32),
            jax.ShapeDtypeStruct((N, 1), jnp.float32),
        ],
    )(q4, zr)


def kernel(z, embeddings):
    zr = z.reshape(N, G, KD, T)
    zb = zr.astype(jnp.bfloat16)
    embt, aug = _prep(embeddings)
    idx = _argmin(zb, aug)
    q = _gather(embt.reshape(G * C, KD), idx.reshape(B))
    out4, loss = _assemble(q.reshape(G, N, T, KD), zr)
    q_merge = out4.reshape(N, 128, 4096)
    vq_loss = loss[:, 0] * (0.25 / (KD * T * G))
    return (q_merge, vq_loss)


# speedup vs baseline: 2.6454x; 1.0496x over previous
import functools

import jax
import jax.numpy as jnp
from jax import lax
from jax.experimental import pallas as pl
from jax.experimental.pallas import tpu as pltpu
from jax.experimental.pallas import tpu_sc as plsc

G = 8
KD = 64
C = 8192
N = 16
T = 1024
CT = 1024
NCT = C // CT
B = G * N * T


KA = 72


def _prep_body(emb_ref, embt_ref, aug_ref):
    e = emb_ref[0]
    et = e.T
    embt_ref[0] = et
    etm2 = (et * -2.0).astype(jnp.bfloat16)
    e2 = jnp.sum(et * et, axis=1, keepdims=True)
    h1 = e2.astype(jnp.bfloat16)
    r1 = e2 - h1.astype(jnp.float32)
    h2 = r1.astype(jnp.bfloat16)
    r2 = r1 - h2.astype(jnp.float32)
    h3 = r2.astype(jnp.bfloat16)
    zpad = jnp.zeros((CT, KA - KD - 3), jnp.bfloat16)
    aug_ref[0] = jnp.concatenate([etm2, h1, h2, h3, zpad], axis=1)


def _prep(embeddings):
    return pl.pallas_call(
        _prep_body,
        grid=(G, NCT),
        in_specs=[pl.BlockSpec((1, KD, CT), lambda g, c: (g, 0, c))],
        out_specs=[
            pl.BlockSpec((1, CT, KD), lambda g, c: (g, c, 0)),
            pl.BlockSpec((1, CT, KA), lambda g, c: (g, c, 0)),
        ],
        out_shape=[
            jax.ShapeDtypeStruct((G, C, KD), jnp.float32),
            jax.ShapeDtypeStruct((G, C, KA), jnp.bfloat16),
        ],
    )(embeddings)


NCH = 8
CH = C // NCH


def _argmin_body(x_ref, aug_ref, idx_ref):
    g = pl.program_id(0)
    xb = x_ref[0, 0]
    onepad = jnp.where(
        lax.broadcasted_iota(jnp.int32, (KA - KD, T), 0) < 3,
        1.0, 0.0).astype(jnp.bfloat16)
    xa = jnp.concatenate([xb, onepad], axis=0)
    s = lax.dot_general(aug_ref[0], xa, (((1,), (0,)), ((), ())),
                        preferred_element_type=jnp.float32)
    idx_ref[0, 0, 0, :] = jnp.argmin(s, axis=0).astype(jnp.int32) + g * C


def _argmin(zb, aug):
    return pl.pallas_call(
        _argmin_body,
        grid=(G, N),
        in_specs=[
            pl.BlockSpec((1, 1, KD, T), lambda g, n: (n, g, 0, 0)),
            pl.BlockSpec((1, C, KA), lambda g, n: (g, 0, 0)),
        ],
        out_specs=pl.BlockSpec((1, 1, 1, T), lambda g, n: (g, n, 0, 0)),
        out_shape=jax.ShapeDtypeStruct((G, N, 1, T), jnp.int32),
    )(zb, aug)


_NW = 32
_BPW = B // _NW
_CH = 128


def _gather(table, idx_flat):
    mesh = plsc.VectorSubcoreMesh(core_axis_name="c", subcore_axis_name="s")

    @functools.partial(
        pl.kernel,
        mesh=mesh,
        compiler_params=pltpu.CompilerParams(use_tc_tiling_on_sc=False),
        out_type=jax.ShapeDtypeStruct((B, KD), jnp.float32),
        scratch_types=[
            pltpu.VMEM((_CH,), jnp.int32),
            pltpu.VMEM((_CH, KD), jnp.float32),
            pltpu.SemaphoreType.DMA,
        ],
    )
    def gk(tab, idx, out, idx_v, rows_v, sem):
        wid = lax.axis_index("s") * 2 + lax.axis_index("c")
        base = wid * _BPW

        def body(i, carry):
            off = base + i * _CH
            pltpu.sync_copy(idx.at[pl.ds(off, _CH)], idx_v)
            pltpu.async_copy(tab.at[idx_v], rows_v, sem).wait()
            pltpu.sync_copy(rows_v, out.at[pl.ds(off, _CH)])
            return carry

        lax.fori_loop(0, _BPW // _CH, body, 0)

    return gk(table, idx_flat)


def _asm_body(q_ref, x_ref, out_ref, loss_ref):
    nidx = pl.program_id(0)
    gidx = pl.program_id(1)
    q = q_ref[0, 0]
    x = x_ref[0, 0]
    qt = q.T
    d = qt - x
    out_ref[0, 0] = x + d

    @pl.when(gidx == 0)
    def _():
        loss_ref[nidx, 0] = 0.0

    loss_ref[nidx, 0] += jnp.sum(d * d)


def _assemble(q4, zr):
    return pl.pallas_call(
        _asm_body,
        grid=(N, G),
        in_specs=[
            pl.BlockSpec((1, 1, T, KD), lambda n, g: (g, n, 0, 0)),
            pl.BlockSpec((1, 1, KD, T), lambda n, g: (n, g, 0, 0)),
        ],
        out_specs=[
            pl.BlockSpec((1, 1, KD, T), lambda n, g: (n, g, 0, 0)),
            pl.BlockSpec((N, 1), lambda n, g: (0, 0),
                         memory_space=pltpu.SMEM),
        ],
        out_shape=[
            jax.ShapeDtypeStruct((N, G, KD, T), jnp.float32),
            jax.ShapeDtypeStruct((N, 1), jnp.float32),
        ],
    )(q4, zr)


def kernel(z, embeddings):
    zr = z.reshape(N, G, KD, T)
    zb = zr.astype(jnp.bfloat16)
    embt, aug = _prep(embeddings)
    idx = _argmin(zb, aug)
    q = _gather(embt.reshape(G * C, KD), idx.reshape(B))
    out4, loss = _assemble(q.reshape(G, N, T, KD), zr)
    q_merge = out4.reshape(N, 128, 4096)
    vq_loss = loss[:, 0] * (0.25 / (KD * T * G))
    return (q_merge, vq_loss)


# speedup vs baseline: 2.6903x; 1.0170x over previous
import functools

import jax
import jax.numpy as jnp
from jax import lax
from jax.experimental import pallas as pl
from jax.experimental.pallas import tpu as pltpu
from jax.experimental.pallas import tpu_sc as plsc

G = 8
KD = 64
C = 8192
N = 16
T = 1024
CT = 1024
NCT = C // CT
B = G * N * T


KA = 72


def _prep_body(emb_ref, embt_ref, aug_ref):
    e = emb_ref[0]
    et = e.T
    embt_ref[0] = et
    etm2 = (et * -2.0).astype(jnp.bfloat16)
    e2 = jnp.sum(et * et, axis=1, keepdims=True)
    h1 = e2.astype(jnp.bfloat16)
    r1 = e2 - h1.astype(jnp.float32)
    h2 = r1.astype(jnp.bfloat16)
    r2 = r1 - h2.astype(jnp.float32)
    h3 = r2.astype(jnp.bfloat16)
    zpad = jnp.zeros((CT, KA - KD - 3), jnp.bfloat16)
    aug_ref[0] = jnp.concatenate([etm2, h1, h2, h3, zpad], axis=1)


def _prep(embeddings):
    return pl.pallas_call(
        _prep_body,
        grid=(G, NCT),
        in_specs=[pl.BlockSpec((1, KD, CT), lambda g, c: (g, 0, c))],
        out_specs=[
            pl.BlockSpec((1, CT, KD), lambda g, c: (g, c, 0)),
            pl.BlockSpec((1, CT, KA), lambda g, c: (g, c, 0)),
        ],
        out_shape=[
            jax.ShapeDtypeStruct((G, C, KD), jnp.float32),
            jax.ShapeDtypeStruct((G, C, KA), jnp.bfloat16),
        ],
    )(embeddings)


NCH = 8
CH = C // NCH


def _argmin_body(x_ref, aug_ref, idx_ref):
    g = pl.program_id(0)
    xb = x_ref[0, 0].astype(jnp.bfloat16)
    onepad = jnp.where(
        lax.broadcasted_iota(jnp.int32, (KA - KD, T), 0) < 3,
        1.0, 0.0).astype(jnp.bfloat16)
    xa = jnp.concatenate([xb, onepad], axis=0)
    s = lax.dot_general(aug_ref[0], xa, (((1,), (0,)), ((), ())),
                        preferred_element_type=jnp.float32)
    idx_ref[0, 0, 0, :] = jnp.argmin(s, axis=0).astype(jnp.int32) + g * C


def _argmin(zr, aug):
    return pl.pallas_call(
        _argmin_body,
        grid=(G, N),
        in_specs=[
            pl.BlockSpec((1, 1, KD, T), lambda g, n: (n, g, 0, 0)),
            pl.BlockSpec((1, C, KA), lambda g, n: (g, 0, 0)),
        ],
        out_specs=pl.BlockSpec((1, 1, 1, T), lambda g, n: (g, n, 0, 0)),
        out_shape=jax.ShapeDtypeStruct((G, N, 1, T), jnp.int32),
    )(zr, aug)


_NW = 32
_BPW = B // _NW
_CH = 128


def _gather(table, idx_flat):
    mesh = plsc.VectorSubcoreMesh(core_axis_name="c", subcore_axis_name="s")

    @functools.partial(
        pl.kernel,
        mesh=mesh,
        compiler_params=pltpu.CompilerParams(use_tc_tiling_on_sc=False),
        out_type=jax.ShapeDtypeStruct((B, KD), jnp.float32),
        scratch_types=[
            pltpu.VMEM((_CH,), jnp.int32),
            pltpu.VMEM((_CH, KD), jnp.float32),
            pltpu.SemaphoreType.DMA,
        ],
    )
    def gk(tab, idx, out, idx_v, rows_v, sem):
        wid = lax.axis_index("s") * 2 + lax.axis_index("c")
        base = wid * _BPW

        def body(i, carry):
            off = base + i * _CH
            pltpu.sync_copy(idx.at[pl.ds(off, _CH)], idx_v)
            pltpu.async_copy(tab.at[idx_v], rows_v, sem).wait()
            pltpu.sync_copy(rows_v, out.at[pl.ds(off, _CH)])
            return carry

        lax.fori_loop(0, _BPW // _CH, body, 0)

    return gk(table, idx_flat)


def _asm_body(q_ref, x_ref, out_ref, loss_ref):
    nidx = pl.program_id(0)
    gidx = pl.program_id(1)
    q = q_ref[0, 0]
    x = x_ref[0, 0]
    qt = q.T
    d = qt - x
    out_ref[0, 0] = x + d

    @pl.when(gidx == 0)
    def _():
        loss_ref[nidx, 0] = 0.0

    loss_ref[nidx, 0] += jnp.sum(d * d)


def _assemble(q4, zr):
    return pl.pallas_call(
        _asm_body,
        grid=(N, G),
        in_specs=[
            pl.BlockSpec((1, 1, T, KD), lambda n, g: (g, n, 0, 0)),
            pl.BlockSpec((1, 1, KD, T), lambda n, g: (n, g, 0, 0)),
        ],
        out_specs=[
            pl.BlockSpec((1, 1, KD, T), lambda n, g: (n, g, 0, 0)),
            pl.BlockSpec((N, 1), lambda n, g: (0, 0),
                         memory_space=pltpu.SMEM),
        ],
        out_shape=[
            jax.ShapeDtypeStruct((N, G, KD, T), jnp.float32),
            jax.ShapeDtypeStruct((N, 1), jnp.float32),
        ],
    )(q4, zr)


def kernel(z, embeddings):
    zr = z.reshape(N, G, KD, T)
    embt, aug = _prep(embeddings)
    idx = _argmin(zr, aug)
    q = _gather(embt.reshape(G * C, KD), idx.reshape(B))
    out4, loss = _assemble(q.reshape(G, N, T, KD), zr)
    q_merge = out4.reshape(N, 128, 4096)
    vq_loss = loss[:, 0] * (0.25 / (KD * T * G))
    return (q_merge, vq_loss)


# speedup vs baseline: 2.8438x; 1.0571x over previous
import functools

import jax
import jax.numpy as jnp
from jax import lax
from jax.experimental import pallas as pl
from jax.experimental.pallas import tpu as pltpu
from jax.experimental.pallas import tpu_sc as plsc

G = 8
KD = 64
C = 8192
N = 16
T = 1024
CT = 1024
NCT = C // CT
B = G * N * T


KA = 72


def _prep_body(emb_ref, embt_ref, aug_ref):
    e = emb_ref[0]
    et = e.T
    embt_ref[0] = jnp.concatenate(
        [et, jnp.zeros((CT, 128 - KD), jnp.float32)], axis=1)
    etm2 = (et * -2.0).astype(jnp.bfloat16)
    e2 = jnp.sum(et * et, axis=1, keepdims=True)
    h1 = e2.astype(jnp.bfloat16)
    r1 = e2 - h1.astype(jnp.float32)
    h2 = r1.astype(jnp.bfloat16)
    r2 = r1 - h2.astype(jnp.float32)
    h3 = r2.astype(jnp.bfloat16)
    zpad = jnp.zeros((CT, KA - KD - 3), jnp.bfloat16)
    aug_ref[0] = jnp.concatenate([etm2, h1, h2, h3, zpad], axis=1)


def _prep(embeddings):
    return pl.pallas_call(
        _prep_body,
        grid=(G, NCT),
        in_specs=[pl.BlockSpec((1, KD, CT), lambda g, c: (g, 0, c))],
        out_specs=[
            pl.BlockSpec((1, CT, 128), lambda g, c: (g, c, 0)),
            pl.BlockSpec((1, CT, KA), lambda g, c: (g, c, 0)),
        ],
        out_shape=[
            jax.ShapeDtypeStruct((G, C, 128), jnp.float32),
            jax.ShapeDtypeStruct((G, C, KA), jnp.bfloat16),
        ],
    )(embeddings)


NCH = 8
CH = C // NCH


def _argmin_body(x_ref, aug_ref, idx_ref):
    g = pl.program_id(0)
    xb = x_ref[0, 0].astype(jnp.bfloat16)
    onepad = jnp.where(
        lax.broadcasted_iota(jnp.int32, (KA - KD, T), 0) < 3,
        1.0, 0.0).astype(jnp.bfloat16)
    xa = jnp.concatenate([xb, onepad], axis=0)
    s = lax.dot_general(aug_ref[0], xa, (((1,), (0,)), ((), ())),
                        preferred_element_type=jnp.float32)
    idx_ref[0, 0, 0, :] = jnp.argmin(s, axis=0).astype(jnp.int32) + g * C


def _argmin(zr, aug):
    return pl.pallas_call(
        _argmin_body,
        grid=(G, N),
        in_specs=[
            pl.BlockSpec((1, 1, KD, T), lambda g, n: (n, g, 0, 0)),
            pl.BlockSpec((1, C, KA), lambda g, n: (g, 0, 0)),
        ],
        out_specs=pl.BlockSpec((1, 1, 1, T), lambda g, n: (g, n, 0, 0)),
        out_shape=jax.ShapeDtypeStruct((G, N, 1, T), jnp.int32),
    )(zr, aug)


_NW = 32
_BPW = B // _NW
_CH = 128


def _gather(table, idx_flat):
    mesh = plsc.VectorSubcoreMesh(core_axis_name="c", subcore_axis_name="s")

    @functools.partial(
        pl.kernel,
        mesh=mesh,
        out_type=jax.ShapeDtypeStruct((B, 128), jnp.float32),
        scratch_types=[
            pltpu.VMEM((_CH,), jnp.int32),
            pltpu.VMEM((_CH, 128), jnp.float32),
            pltpu.SemaphoreType.DMA,
        ],
    )
    def gk(tab, idx, out, idx_v, rows_v, sem):
        wid = lax.axis_index("s") * 2 + lax.axis_index("c")
        base = wid * _BPW

        def body(i, carry):
            off = base + i * _CH
            pltpu.sync_copy(idx.at[pl.ds(off, _CH)], idx_v)
            pltpu.async_copy(tab.at[idx_v], rows_v, sem).wait()
            pltpu.sync_copy(rows_v, out.at[pl.ds(off, _CH)])
            return carry

        lax.fori_loop(0, _BPW // _CH, body, 0)

    return gk(table, idx_flat)


def _asm_body(q_ref, x_ref, out_ref, loss_ref):
    nidx = pl.program_id(0)
    gidx = pl.program_id(1)
    q = q_ref[0, 0, :, :KD]
    x = x_ref[0, 0]
    qt = q.T
    d = qt - x
    out_ref[0, 0] = x + d

    @pl.when(gidx == 0)
    def _():
        loss_ref[nidx, 0] = 0.0

    loss_ref[nidx, 0] += jnp.sum(d * d)


def _assemble(q4, zr):
    return pl.pallas_call(
        _asm_body,
        grid=(N, G),
        in_specs=[
            pl.BlockSpec((1, 1, T, 128), lambda n, g: (g, n, 0, 0)),
            pl.BlockSpec((1, 1, KD, T), lambda n, g: (n, g, 0, 0)),
        ],
        out_specs=[
            pl.BlockSpec((1, 1, KD, T), lambda n, g: (n, g, 0, 0)),
            pl.BlockSpec((N, 1), lambda n, g: (0, 0),
                         memory_space=pltpu.SMEM),
        ],
        out_shape=[
            jax.ShapeDtypeStruct((N, G, KD, T), jnp.float32),
            jax.ShapeDtypeStruct((N, 1), jnp.float32),
        ],
    )(q4, zr)


def kernel(z, embeddings):
    zr = z.reshape(N, G, KD, T)
    embt, aug = _prep(embeddings)
    idx = _argmin(zr, aug)
    q = _gather(embt.reshape(G * C, 128), idx.reshape(B))
    out4, loss = _assemble(q.reshape(G, N, T, 128), zr)
    q_merge = out4.reshape(N, 128, 4096)
    vq_loss = loss[:, 0] * (0.25 / (KD * T * G))
    return (q_merge, vq_loss)


# speedup vs baseline: 2.9405x; 1.0340x over previous
import functools

import jax
import jax.numpy as jnp
from jax import lax
from jax.experimental import pallas as pl
from jax.experimental.pallas import tpu as pltpu
from jax.experimental.pallas import tpu_sc as plsc

G = 8
KD = 64
C = 8192
N = 16
T = 1024
CT = 1024
NCT = C // CT
B = G * N * T


KA = 72


def _prep_body(emb_ref, embt_ref, aug_ref):
    e = emb_ref[0]
    et = e.T
    embt_ref[0] = jnp.concatenate(
        [et, jnp.zeros((CT, 128 - KD), jnp.float32)], axis=1)
    etm2 = (et * -2.0).astype(jnp.bfloat16)
    e2 = jnp.sum(et * et, axis=1, keepdims=True)
    h1 = e2.astype(jnp.bfloat16)
    r1 = e2 - h1.astype(jnp.float32)
    h2 = r1.astype(jnp.bfloat16)
    r2 = r1 - h2.astype(jnp.float32)
    h3 = r2.astype(jnp.bfloat16)
    zpad = jnp.zeros((CT, KA - KD - 3), jnp.bfloat16)
    aug_ref[0] = jnp.concatenate([etm2, h1, h2, h3, zpad], axis=1)


def _prep(embeddings):
    return pl.pallas_call(
        _prep_body,
        grid=(G, NCT),
        in_specs=[pl.BlockSpec((1, KD, CT), lambda g, c: (g, 0, c))],
        out_specs=[
            pl.BlockSpec((1, CT, 128), lambda g, c: (g, c, 0)),
            pl.BlockSpec((1, CT, KA), lambda g, c: (g, c, 0)),
        ],
        out_shape=[
            jax.ShapeDtypeStruct((G, C, 128), jnp.float32),
            jax.ShapeDtypeStruct((G, C, KA), jnp.bfloat16),
        ],
    )(embeddings)


NCH = 8
CH = C // NCH


def _argmin_body(x_ref, aug_ref, idx_ref):
    g = pl.program_id(0)
    xb = x_ref[0, 0].astype(jnp.bfloat16)
    onepad = jnp.where(
        lax.broadcasted_iota(jnp.int32, (KA - KD, T), 0) < 3,
        1.0, 0.0).astype(jnp.bfloat16)
    xa = jnp.concatenate([xb, onepad], axis=0)
    s = lax.dot_general(aug_ref[0], xa, (((1,), (0,)), ((), ())),
                        preferred_element_type=jnp.float32)
    idx_ref[0, 0, 0, :] = jnp.argmin(s, axis=0).astype(jnp.int32) + g * C


def _argmin(zr, aug):
    return pl.pallas_call(
        _argmin_body,
        grid=(G, N),
        in_specs=[
            pl.BlockSpec((1, 1, KD, T), lambda g, n: (n, g, 0, 0)),
            pl.BlockSpec((1, C, KA), lambda g, n: (g, 0, 0)),
        ],
        out_specs=pl.BlockSpec((1, 1, 1, T), lambda g, n: (g, n, 0, 0)),
        out_shape=jax.ShapeDtypeStruct((G, N, 1, T), jnp.int32),
    )(zr, aug)


_NW = 32
_BPW = B // _NW
_CH = 128


_NB = 4
_NCHK = _BPW // _CH


def _gather(table, idx2d):
    mesh = plsc.VectorSubcoreMesh(core_axis_name="c", subcore_axis_name="s")

    @functools.partial(
        pl.kernel,
        mesh=mesh,
        out_type=jax.ShapeDtypeStruct((B, 128), jnp.float32),
        scratch_types=[
            pltpu.VMEM((_NCHK, _CH), jnp.int32),
            [pltpu.VMEM((_CH, 128), jnp.float32) for _ in range(_NB)],
            [pltpu.SemaphoreType.DMA for _ in range(_NB)],
        ],
    )
    def gk(tab, idx, out, idx_v, bufs, sems):
        wid = lax.axis_index("s") * 2 + lax.axis_index("c")
        pltpu.sync_copy(idx.at[pl.ds(wid * _NCHK, _NCHK)], idx_v)
        base = wid * _NCHK

        def body(j, carry):
            b = j * _NB
            cps = [
                pltpu.async_copy(tab.at[idx_v.at[b + k]], bufs[k], sems[k])
                for k in range(_NB)
            ]
            for k in range(_NB):
                cps[k].wait()
                pltpu.sync_copy(
                    bufs[k], out.at[pl.ds((base + b + k) * _CH, _CH)])
            return carry

        lax.fori_loop(0, _NCHK // _NB, body, 0)

    return gk(table, idx2d)


def _asm_body(q_ref, x_ref, out_ref, loss_ref):
    nidx = pl.program_id(0)
    gidx = pl.program_id(1)
    q = q_ref[0, 0, :, :KD]
    x = x_ref[0, 0]
    qt = q.T
    d = qt - x
    out_ref[0, 0] = x + d

    @pl.when(gidx == 0)
    def _():
        loss_ref[nidx, 0] = 0.0

    loss_ref[nidx, 0] += jnp.sum(d * d)


def _assemble(q4, zr):
    return pl.pallas_call(
        _asm_body,
        grid=(N, G),
        in_specs=[
            pl.BlockSpec((1, 1, T, 128), lambda n, g: (g, n, 0, 0)),
            pl.BlockSpec((1, 1, KD, T), lambda n, g: (n, g, 0, 0)),
        ],
        out_specs=[
            pl.BlockSpec((1, 1, KD, T), lambda n, g: (n, g, 0, 0)),
            pl.BlockSpec((N, 1), lambda n, g: (0, 0),
                         memory_space=pltpu.SMEM),
        ],
        out_shape=[
            jax.ShapeDtypeStruct((N, G, KD, T), jnp.float32),
            jax.ShapeDtypeStruct((N, 1), jnp.float32),
        ],
    )(q4, zr)


def kernel(z, embeddings):
    zr = z.reshape(N, G, KD, T)
    embt, aug = _prep(embeddings)
    idx = _argmin(zr, aug)
    q = _gather(embt.reshape(G * C, 128), idx.reshape(B // _CH, _CH))
    out4, loss = _assemble(q.reshape(G, N, T, 128), zr)
    q_merge = out4.reshape(N, 128, 4096)
    vq_loss = loss[:, 0] * (0.25 / (KD * T * G))
    return (q_merge, vq_loss)
